# baseline (speedup 1.0000x reference)
import jax
import jax.numpy as jnp
from jax.experimental import pallas as pl

_DEPTH = 2000
_DBLK = 400


def _onehot_block(idx_ref, out_ref):
    j = pl.program_id(0)
    idx = idx_ref[0, j, :]
    d0 = pl.program_id(1) * _DBLK
    iota = jax.lax.broadcasted_iota(jnp.int32, (_DBLK, idx.shape[0]), 0)
    out_ref[0] = (iota == (idx - d0)[None, :]).astype(jnp.float32)


def kernel(inputs):
    n, m = inputs.shape
    idx_t = inputs.astype(jnp.int32).T.reshape(1, m, n)
    out = pl.pallas_call(
        _onehot_block,
        grid=(m, _DEPTH // _DBLK),
        in_specs=[pl.BlockSpec((1, m, n), lambda j, k: (0, 0, 0))],
        out_specs=pl.BlockSpec((1, _DBLK, n), lambda j, k: (j, k, 0)),
        out_shape=jax.ShapeDtypeStruct((m, _DEPTH, n), jnp.float32),
    )(idx_t)
    return out.transpose(2, 0, 1)


# speedup vs baseline: 1.2849x; 1.2849x over previous
import jax
import jax.numpy as jnp
from jax.experimental import pallas as pl

_DEPTH = 2000
_DBLK = 1000


def _onehot_block(idx_ref, out_ref):
    j = pl.program_id(0)
    idx = idx_ref[0, j, :]
    d0 = pl.program_id(1) * _DBLK
    iota = jax.lax.broadcasted_iota(jnp.int32, (_DBLK, idx.shape[0]), 0)
    out_ref[0] = (iota == (idx - d0)[None, :]).astype(jnp.float32)


def kernel(inputs):
    n, m = inputs.shape
    idx_t = inputs.astype(jnp.int32).T.reshape(1, m, n)
    out = pl.pallas_call(
        _onehot_block,
        grid=(m, _DEPTH // _DBLK),
        in_specs=[pl.BlockSpec((1, m, n), lambda j, k: (0, 0, 0))],
        out_specs=pl.BlockSpec((1, _DBLK, n), lambda j, k: (j, k, 0)),
        out_shape=jax.ShapeDtypeStruct((m, _DEPTH, n), jnp.float32),
    )(idx_t)
    return out.transpose(2, 0, 1)
